# baseline (speedup 1.0000x reference)
import functools

import jax
import jax.numpy as jnp
from jax import lax
from jax.experimental import pallas as pl
from jax.experimental.pallas import tpu as pltpu
from jax.experimental.pallas import tpu_sc as plsc

_B = 4096
_D = 256
_NC = 2
_NS = 16
_L = 16
_NW = _NC * _NS
_BPW = _B // _NW
_CHUNKS = _D // _L

_mesh = plsc.VectorSubcoreMesh(core_axis_name="c", subcore_axis_name="s")


@functools.partial(
    pl.kernel,
    mesh=_mesh,
    out_type=jax.ShapeDtypeStruct((_NW, _L), jnp.float32),
    scratch_types=[
        pltpu.VMEM((_BPW,), jnp.int32),
        pltpu.VMEM((_BPW, _D), jnp.float32),
        pltpu.VMEM((_BPW, _D), jnp.float32),
        pltpu.VMEM((_L,), jnp.float32),
        pltpu.SemaphoreType.DMA,
    ],
)
def _center_loss_partials(feat_hbm, lab_hbm, cent_hbm, out_hbm,
                          idx_v, cent_v, feat_v, acc_v, sem):
    wid = lax.axis_index("s") * _NC + lax.axis_index("c")
    base = wid * _BPW
    pltpu.sync_copy(lab_hbm.at[pl.ds(base, _BPW)], idx_v)
    gather = pltpu.async_copy(cent_hbm.at[idx_v], cent_v, sem)
    pltpu.sync_copy(feat_hbm.at[pl.ds(base, _BPW)], feat_v)
    gather.wait()

    def body(r, accs):
        new = list(accs)
        for c in range(_CHUNKS):
            f = feat_v[r, pl.ds(c * _L, _L)]
            g = cent_v[r, pl.ds(c * _L, _L)]
            d = f - g
            new[c % 4] = new[c % 4] + d * d
        return tuple(new)

    zero = jnp.zeros((_L,), jnp.float32)
    a0, a1, a2, a3 = lax.fori_loop(0, _BPW, body, (zero, zero, zero, zero))
    acc_v[...] = (a0 + a1) + (a2 + a3)
    pltpu.sync_copy(acc_v, out_hbm.at[wid])


def kernel(features, labels, centers):
    partials = _center_loss_partials(features, labels.astype(jnp.int32), centers)
    return jnp.sum(partials) / jnp.float32(_B * _D)
